# baseline (speedup 1.0000x reference)
import jax
import jax.numpy as jnp
from jax.experimental import pallas as pl
from jax.experimental.pallas import tpu as pltpu

EMB = 768
BLK = 64
NC = 97
NKB = EMB // BLK
KW = BLK * BLK
NBUF = 3


def _wb_copy(wb_hbm, wb_scr, wb_sem, k):
    return pltpu.make_async_copy(
        wb_hbm.at[:, k * KW:(k + 1) * KW], wb_scr.at[k % NBUF],
        wb_sem.at[k % NBUF])


def _doc_kernel(pos_ref, ht_ref, x_ref, a_ref, wh_ref, bh_ref, bb_ref,
                wb_hbm, out_ref, hs_scr, ts_scr, s_scr, ent_scr, q_scr,
                wb_scr, wb_sem):
    f32 = jnp.float32
    bf16 = jnp.bfloat16
    d = pl.program_id(0)
    h = pl.program_id(1)
    B = pl.num_programs(0)
    NH = pl.num_programs(1)
    pos = pos_ref[0]
    ht = ht_ref[0]
    NE, M = pos.shape
    NR = ht.shape[0]
    L = x_ref.shape[1]
    NRP = hs_scr.shape[1] // B

    @pl.when(jnp.logical_and(d == 0, h == 0))
    def _prologue():
        hs_scr[...] = jnp.zeros(hs_scr.shape, bf16)
        ts_scr[...] = jnp.zeros(ts_scr.shape, bf16)
        for k in range(NBUF):
            _wb_copy(wb_hbm, wb_scr, wb_sem, k).start()

    @pl.when(h == 0)
    def _doc_head():
        li = jax.lax.broadcasted_iota(jnp.int32, (NE, M, L), 2)
        s_scr[...] = (pos[:, :, None] == li).astype(bf16).sum(axis=1)
        EX = jnp.exp(x_ref[0]).astype(bf16)
        ent_scr[...] = jnp.log(jax.lax.dot_general(
            EX, s_scr[...], (((0,), (1,)), ((), ())),
            preferred_element_type=f32)).astype(bf16)

    S = s_scr[...]
    EAh = jax.lax.dot_general(S, a_ref[0, 0], (((1,), (0,)), ((), ())),
                              preferred_element_type=f32)
    EAhb = EAh.astype(bf16)
    Qh = EAhb[:, None, :] * EAhb[None, :, :]

    @pl.when(h == 0)
    def _q_init():
        q_scr[...] = Qh.astype(f32)

    @pl.when(h > 0)
    def _q_acc():
        q_scr[...] = q_scr[...] + Qh

    @pl.when(h == NH - 1)
    def _doc_tail():
        X = x_ref[0]
        Qr = q_scr[...].reshape(NE * NE, L).astype(bf16)
        ci = ht[:, 0] * NE + ht[:, 1]
        qi = jax.lax.broadcasted_iota(jnp.int32, (NR, NE * NE), 1)
        OC = (ci[:, None] == qi).astype(bf16)
        rsT = jax.lax.dot_general(Qr, OC, (((0,), (1,)), ((), ())),
                                  preferred_element_type=f32)
        rsT = rsT / jnp.sum(rsT, axis=0, keepdims=True)
        rdocT = jax.lax.dot_general(X.astype(bf16), rsT.astype(bf16),
                                    (((0,), (0,)), ((), ())),
                                    preferred_element_type=f32)
        ei = jax.lax.broadcasted_iota(jnp.int32, (NR, NE), 1)
        OH = (ht[:, 0][:, None] == ei).astype(bf16)
        OT = (ht[:, 1][:, None] == ei).astype(bf16)
        entTb = ent_scr[...]
        hsT_e = jax.lax.dot_general(entTb, OH, (((1,), (1,)), ((), ())),
                                    preferred_element_type=f32)
        tsT_e = jax.lax.dot_general(entTb, OT, (((1,), (1,)), ((), ())),
                                    preferred_element_type=f32)
        W1 = wh_ref[:, :EMB].astype(bf16)
        W2 = wh_ref[:, EMB:].astype(bf16)
        b = bh_ref[...]
        ctx = jax.lax.dot_general(W2, rdocT.astype(bf16),
                                  (((1,), (0,)), ((), ())),
                                  preferred_element_type=f32) + b
        hsT = jnp.tanh(
            jax.lax.dot_general(W1, hsT_e.astype(bf16),
                                (((1,), (0,)), ((), ())),
                                preferred_element_type=f32)
            + ctx).astype(bf16)
        tsT = jnp.tanh(
            jax.lax.dot_general(W1, tsT_e.astype(bf16),
                                (((1,), (0,)), ((), ())),
                                preferred_element_type=f32)
            + ctx).astype(bf16)
        for dd in range(B):
            @pl.when(d == dd)
            def _store(dd=dd):
                hs_scr[:, dd * NRP:dd * NRP + NR] = hsT
                ts_scr[:, dd * NRP:dd * NRP + NR] = tsT

    @pl.when(jnp.logical_and(d == B - 1, h == NH - 1))
    def _bilinear():
        hsA = hs_scr[...]
        tsA = ts_scr[...]
        NT = hsA.shape[1]
        acc = jnp.zeros((NC, NT), f32)
        for k in range(NKB):
            _wb_copy(wb_hbm, wb_scr, wb_sem, k).wait()
            wbk = wb_scr[k % NBUF].astype(bf16)
            hk = hsA[k * BLK:(k + 1) * BLK, :]
            tk = tsA[k * BLK:(k + 1) * BLK, :]
            b3 = hk[:, None, :] * tk[None, :, :]
            b2 = b3.reshape(KW, NT)
            acc = acc + jax.lax.dot_general(
                wbk, b2, (((1,), (0,)), ((), ())),
                preferred_element_type=f32)
            if k + NBUF < NKB:
                _wb_copy(wb_hbm, wb_scr, wb_sem, k + NBUF).start()
        acc = acc + bb_ref[...]
        for dd in range(B):
            out_ref[dd] = acc[:, dd * NRP:dd * NRP + NR]


def kernel(seq_embs, attentions, entity_pos, hts, n_entities, n_rels,
           W_head, b_head, W_bil, b_bil):
    B, L, Hd = seq_embs.shape
    NH = attentions.shape[1]
    TE = entity_pos.shape[0]
    TR = hts.shape[0]
    NE = TE // B
    M = entity_pos.shape[1]
    NR = TR // B
    NRP = ((NR + 127) // 128) * 128

    pos3 = entity_pos.reshape(B, NE, M)
    hts3 = hts.reshape(B, NR, 2)
    bh = b_head.reshape(EMB, 1)
    bb = b_bil.reshape(NC, 1)

    outT = pl.pallas_call(
        _doc_kernel,
        grid=(B, NH),
        in_specs=[
            pl.BlockSpec((1, NE, M), lambda d, h: (d, 0, 0)),
            pl.BlockSpec((1, NR, 2), lambda d, h: (d, 0, 0)),
            pl.BlockSpec((1, L, Hd), lambda d, h: (d, 0, 0)),
            pl.BlockSpec((1, 1, L, L), lambda d, h: (d, h, 0, 0)),
            pl.BlockSpec((EMB, 2 * Hd), lambda d, h: (0, 0)),
            pl.BlockSpec((EMB, 1), lambda d, h: (0, 0)),
            pl.BlockSpec((NC, 1), lambda d, h: (0, 0)),
            pl.BlockSpec(memory_space=pltpu.MemorySpace.HBM),
        ],
        out_specs=pl.BlockSpec((B, NC, NR), lambda d, h: (0, 0, 0)),
        out_shape=jax.ShapeDtypeStruct((B, NC, NR), jnp.float32),
        scratch_shapes=[
            pltpu.VMEM((EMB, B * NRP), jnp.bfloat16),
            pltpu.VMEM((EMB, B * NRP), jnp.bfloat16),
            pltpu.VMEM((NE, L), jnp.bfloat16),
            pltpu.VMEM((EMB, NE), jnp.bfloat16),
            pltpu.VMEM((NE, NE, L), jnp.float32),
            pltpu.VMEM((NBUF, NC, KW), jnp.float32),
            pltpu.SemaphoreType.DMA((NBUF,)),
        ],
    )(pos3, hts3, seq_embs, attentions, W_head, bh, bb, W_bil)

    return jnp.transpose(outT, (0, 2, 1)).reshape(TR, NC)


# speedup vs baseline: 1.1905x; 1.1905x over previous
import jax
import jax.numpy as jnp
from jax.experimental import pallas as pl
from jax.experimental.pallas import tpu as pltpu

EMB = 768
BLK = 64
NC = 97
NKB = EMB // BLK
KW = BLK * BLK
NBUF = 3
ABUF = 4
ALOOK = 3


def _wb_copy(wb_hbm, wb_scr, wb_sem, k):
    return pltpu.make_async_copy(
        wb_hbm.at[:, k * KW:(k + 1) * KW], wb_scr.at[k % NBUF],
        wb_sem.at[k % NBUF])


def _at_copy(a_hbm, at_scr, at_sem, d, h):
    return pltpu.make_async_copy(
        a_hbm.at[d, h % 12], at_scr.at[h % ABUF], at_sem.at[h % ABUF])


def _doc_kernel(pos_ref, ht_ref, x_ref, bh_ref, bb_ref, a_hbm, wh_hbm,
                wb_hbm, out_ref, hs_scr, ts_scr, wh_scr, wb_scr,
                at_scr, wb_sem, at_sem, wh_sem):
    f32 = jnp.float32
    bf16 = jnp.bfloat16
    d = pl.program_id(0)
    B = pl.num_programs(0)
    pos = pos_ref[0]
    ht = ht_ref[0]
    X = x_ref[0]
    NE, M = pos.shape
    NR = ht.shape[0]
    L = X.shape[0]
    NH = a_hbm.shape[1]
    NRP = hs_scr.shape[1] // B

    @pl.when(d == 0)
    def _prologue():
        hs_scr[...] = jnp.zeros(hs_scr.shape, bf16)
        ts_scr[...] = jnp.zeros(ts_scr.shape, bf16)
        pltpu.make_async_copy(wh_hbm, wh_scr, wh_sem).start()
        for k in range(NBUF):
            _wb_copy(wb_hbm, wb_scr, wb_sem, k).start()
        for h in range(ALOOK):
            _at_copy(a_hbm, at_scr, at_sem, d, h).start()

    li = jax.lax.broadcasted_iota(jnp.int32, (NE, M, L), 2)
    S = (pos[:, :, None] == li).astype(bf16).sum(axis=1)

    EX = jnp.exp(X).astype(bf16)
    entT = jnp.log(jax.lax.dot_general(
        EX, S, (((0,), (1,)), ((), ())),
        preferred_element_type=f32))

    Q = jnp.zeros((NE, NE, L), f32)
    for h in range(NH):
        _at_copy(a_hbm, at_scr, at_sem, d, h).wait()
        EAh = jax.lax.dot_general(S, at_scr[h % ABUF].astype(bf16),
                                  (((1,), (0,)), ((), ())),
                                  preferred_element_type=f32)
        EAhb = EAh.astype(bf16)
        Q = Q + EAhb[:, None, :] * EAhb[None, :, :]
        nh = h + ALOOK
        if nh < NH:
            _at_copy(a_hbm, at_scr, at_sem, d, nh).start()
        else:
            @pl.when(d + 1 < B)
            def _ahead(nh=nh):
                _at_copy(a_hbm, at_scr, at_sem, d + 1, nh).start()
    Qr = Q.reshape(NE * NE, L).astype(bf16)

    ci = ht[:, 0] * NE + ht[:, 1]
    qi = jax.lax.broadcasted_iota(jnp.int32, (NR, NE * NE), 1)
    OC = (ci[:, None] == qi).astype(bf16)
    rsT = jax.lax.dot_general(Qr, OC, (((0,), (1,)), ((), ())),
                              preferred_element_type=f32)
    rsT = rsT / jnp.sum(rsT, axis=0, keepdims=True)

    rdocT = jax.lax.dot_general(X.astype(bf16), rsT.astype(bf16),
                                (((0,), (0,)), ((), ())),
                                preferred_element_type=f32)

    ei = jax.lax.broadcasted_iota(jnp.int32, (NR, NE), 1)
    OH = (ht[:, 0][:, None] == ei).astype(bf16)
    OT = (ht[:, 1][:, None] == ei).astype(bf16)
    entTb = entT.astype(bf16)
    hsT_e = jax.lax.dot_general(entTb, OH, (((1,), (1,)), ((), ())),
                                preferred_element_type=f32)
    tsT_e = jax.lax.dot_general(entTb, OT, (((1,), (1,)), ((), ())),
                                preferred_element_type=f32)

    @pl.when(d == 0)
    def _wh_wait():
        pltpu.make_async_copy(wh_hbm, wh_scr, wh_sem).wait()
    W1 = wh_scr[:, :EMB].astype(bf16)
    W2 = wh_scr[:, EMB:].astype(bf16)
    b = bh_ref[...]
    ctx = jax.lax.dot_general(W2, rdocT.astype(bf16), (((1,), (0,)), ((), ())),
                              preferred_element_type=f32) + b
    hsT = jnp.tanh(
        jax.lax.dot_general(W1, hsT_e.astype(bf16), (((1,), (0,)), ((), ())),
                            preferred_element_type=f32)
        + ctx).astype(bf16)
    tsT = jnp.tanh(
        jax.lax.dot_general(W1, tsT_e.astype(bf16), (((1,), (0,)), ((), ())),
                            preferred_element_type=f32)
        + ctx).astype(bf16)

    for dd in range(B):
        @pl.when(d == dd)
        def _store(dd=dd):
            hs_scr[:, dd * NRP:dd * NRP + NR] = hsT
            ts_scr[:, dd * NRP:dd * NRP + NR] = tsT

    @pl.when(d == B - 1)
    def _bilinear():
        hsA = hs_scr[...]
        tsA = ts_scr[...]
        NT = hsA.shape[1]
        acc = jnp.zeros((NC, NT), f32)
        for k in range(NKB):
            _wb_copy(wb_hbm, wb_scr, wb_sem, k).wait()
            wbk = wb_scr[k % NBUF].astype(bf16)
            hk = hsA[k * BLK:(k + 1) * BLK, :]
            tk = tsA[k * BLK:(k + 1) * BLK, :]
            b3 = hk[:, None, :] * tk[None, :, :]
            b2 = b3.reshape(KW, NT)
            acc = acc + jax.lax.dot_general(
                wbk, b2, (((1,), (0,)), ((), ())),
                preferred_element_type=f32)
            if k + NBUF < NKB:
                _wb_copy(wb_hbm, wb_scr, wb_sem, k + NBUF).start()
        acc = acc + bb_ref[...]
        for dd in range(B):
            out_ref[dd] = acc[:, dd * NRP:dd * NRP + NR]


def kernel(seq_embs, attentions, entity_pos, hts, n_entities, n_rels,
           W_head, b_head, W_bil, b_bil):
    B, L, Hd = seq_embs.shape
    NH = attentions.shape[1]
    TE = entity_pos.shape[0]
    TR = hts.shape[0]
    NE = TE // B
    M = entity_pos.shape[1]
    NR = TR // B
    NRP = ((NR + 127) // 128) * 128

    pos3 = entity_pos.reshape(B, NE, M)
    hts3 = hts.reshape(B, NR, 2)
    bh = b_head.reshape(EMB, 1)
    bb = b_bil.reshape(NC, 1)

    hbm = pltpu.MemorySpace.HBM
    outT = pl.pallas_call(
        _doc_kernel,
        grid=(B,),
        in_specs=[
            pl.BlockSpec((1, NE, M), lambda d: (d, 0, 0)),
            pl.BlockSpec((1, NR, 2), lambda d: (d, 0, 0)),
            pl.BlockSpec((1, L, Hd), lambda d: (d, 0, 0)),
            pl.BlockSpec((EMB, 1), lambda d: (0, 0)),
            pl.BlockSpec((NC, 1), lambda d: (0, 0)),
            pl.BlockSpec(memory_space=hbm),
            pl.BlockSpec(memory_space=hbm),
            pl.BlockSpec(memory_space=hbm),
        ],
        out_specs=pl.BlockSpec((B, NC, NR), lambda d: (0, 0, 0)),
        out_shape=jax.ShapeDtypeStruct((B, NC, NR), jnp.float32),
        scratch_shapes=[
            pltpu.VMEM((EMB, B * NRP), jnp.bfloat16),
            pltpu.VMEM((EMB, B * NRP), jnp.bfloat16),
            pltpu.VMEM((EMB, 2 * Hd), jnp.float32),
            pltpu.VMEM((NBUF, NC, KW), jnp.float32),
            pltpu.VMEM((ABUF, L, L), jnp.float32),
            pltpu.SemaphoreType.DMA((NBUF,)),
            pltpu.SemaphoreType.DMA((ABUF,)),
            pltpu.SemaphoreType.DMA,
        ],
    )(pos3, hts3, seq_embs, bh, bb, attentions, W_head, W_bil)

    return jnp.transpose(outT, (0, 2, 1)).reshape(TR, NC)


# speedup vs baseline: 1.3190x; 1.1079x over previous
import jax
import jax.numpy as jnp
from jax.experimental import pallas as pl
from jax.experimental.pallas import tpu as pltpu

EMB = 768
BLK = 64
NC = 97
NKB = EMB // BLK
KW = BLK * BLK
NBUF = 3
NSUB = 2


def _wb_copy(wb_hbm, wb_scr, wb_sem, k):
    return pltpu.make_async_copy(
        wb_hbm.at[:, k * KW:(k + 1) * KW], wb_scr.at[k % NBUF],
        wb_sem.at[k % NBUF])


def _doc_kernel(pos_ref, ht_ref, x_ref, a_ref, bh_ref, bb_ref, wh_hbm,
                wb_hbm, out_ref, hs_scr, ts_scr, q_scr, wh_scr, wb_scr,
                wb_sem, wh_sem):
    f32 = jnp.float32
    bf16 = jnp.bfloat16
    d = pl.program_id(0)
    j = pl.program_id(1)
    B = pl.num_programs(0)
    pos = pos_ref[0]
    ht = ht_ref[0]
    NE, M = pos.shape
    NR = ht.shape[0]
    L = x_ref.shape[1]
    NHS = a_ref.shape[1]
    NRP = hs_scr.shape[1] // B

    @pl.when(jnp.logical_and(d == 0, j == 0))
    def _prologue():
        hs_scr[...] = jnp.zeros(hs_scr.shape, bf16)
        ts_scr[...] = jnp.zeros(ts_scr.shape, bf16)
        pltpu.make_async_copy(wh_hbm, wh_scr, wh_sem).start()
        for k in range(NBUF):
            _wb_copy(wb_hbm, wb_scr, wb_sem, k).start()

    li = jax.lax.broadcasted_iota(jnp.int32, (NE, M, L), 2)
    S = (pos[:, :, None] == li).astype(bf16).sum(axis=1)

    Qloc = jnp.zeros((NE, NE, L), f32)
    for h in range(NHS):
        EAh = jax.lax.dot_general(S, a_ref[0, h].astype(bf16),
                                  (((1,), (0,)), ((), ())),
                                  preferred_element_type=f32)
        EAhb = EAh.astype(bf16)
        Qloc = Qloc + EAhb[:, None, :] * EAhb[None, :, :]

    @pl.when(j == 0)
    def _q_init():
        q_scr[...] = Qloc

    @pl.when(j != 0)
    def _doc_tail():
        X = x_ref[0]
        Q = q_scr[...] + Qloc
        EX = jnp.exp(X).astype(bf16)
        entTb = jnp.log(jax.lax.dot_general(
            EX, S, (((0,), (1,)), ((), ())),
            preferred_element_type=f32)).astype(bf16)
        Qr = Q.reshape(NE * NE, L).astype(bf16)
        ci = ht[:, 0] * NE + ht[:, 1]
        qi = jax.lax.broadcasted_iota(jnp.int32, (NR, NE * NE), 1)
        OC = (ci[:, None] == qi).astype(bf16)
        rsT = jax.lax.dot_general(Qr, OC, (((0,), (1,)), ((), ())),
                                  preferred_element_type=f32)
        rsT = rsT / jnp.sum(rsT, axis=0, keepdims=True)
        rdocT = jax.lax.dot_general(X.astype(bf16), rsT.astype(bf16),
                                    (((0,), (0,)), ((), ())),
                                    preferred_element_type=f32)
        ei = jax.lax.broadcasted_iota(jnp.int32, (NR, NE), 1)
        OH = (ht[:, 0][:, None] == ei).astype(bf16)
        OT = (ht[:, 1][:, None] == ei).astype(bf16)
        hsT_e = jax.lax.dot_general(entTb, OH, (((1,), (1,)), ((), ())),
                                    preferred_element_type=f32)
        tsT_e = jax.lax.dot_general(entTb, OT, (((1,), (1,)), ((), ())),
                                    preferred_element_type=f32)
        @pl.when(d == 0)
        def _wh_wait():
            pltpu.make_async_copy(wh_hbm, wh_scr, wh_sem).wait()
        W1 = wh_scr[:, :EMB].astype(bf16)
        W2 = wh_scr[:, EMB:].astype(bf16)
        b = bh_ref[...]
        ctx = jax.lax.dot_general(W2, rdocT.astype(bf16),
                                  (((1,), (0,)), ((), ())),
                                  preferred_element_type=f32) + b
        hsT = jnp.tanh(
            jax.lax.dot_general(W1, hsT_e.astype(bf16),
                                (((1,), (0,)), ((), ())),
                                preferred_element_type=f32)
            + ctx).astype(bf16)
        tsT = jnp.tanh(
            jax.lax.dot_general(W1, tsT_e.astype(bf16),
                                (((1,), (0,)), ((), ())),
                                preferred_element_type=f32)
            + ctx).astype(bf16)
        for dd in range(B):
            @pl.when(d == dd)
            def _store(dd=dd):
                hs_scr[:, dd * NRP:dd * NRP + NR] = hsT
                ts_scr[:, dd * NRP:dd * NRP + NR] = tsT

    @pl.when(jnp.logical_and(d == B - 1, j == NSUB - 1))
    def _bilinear():
        hsA = hs_scr[...]
        tsA = ts_scr[...]
        NT = hsA.shape[1]
        acc = jnp.zeros((NC, NT), f32)
        for k in range(NKB):
            _wb_copy(wb_hbm, wb_scr, wb_sem, k).wait()
            wbk = wb_scr[k % NBUF].astype(bf16)
            hk = hsA[k * BLK:(k + 1) * BLK, :]
            tk = tsA[k * BLK:(k + 1) * BLK, :]
            b3 = hk[:, None, :] * tk[None, :, :]
            b2 = b3.reshape(KW, NT)
            acc = acc + jax.lax.dot_general(
                wbk, b2, (((1,), (0,)), ((), ())),
                preferred_element_type=f32)
            if k + NBUF < NKB:
                _wb_copy(wb_hbm, wb_scr, wb_sem, k + NBUF).start()
        acc = acc + bb_ref[...]
        for dd in range(B):
            out_ref[dd] = acc[:, dd * NRP:dd * NRP + NR]


def kernel(seq_embs, attentions, entity_pos, hts, n_entities, n_rels,
           W_head, b_head, W_bil, b_bil):
    B, L, Hd = seq_embs.shape
    NH = attentions.shape[1]
    TE = entity_pos.shape[0]
    TR = hts.shape[0]
    NE = TE // B
    M = entity_pos.shape[1]
    NR = TR // B
    NRP = ((NR + 127) // 128) * 128
    NHS = NH // NSUB

    pos3 = entity_pos.reshape(B, NE, M)
    hts3 = hts.reshape(B, NR, 2)
    bh = b_head.reshape(EMB, 1)
    bb = b_bil.reshape(NC, 1)

    hbm = pltpu.MemorySpace.HBM
    outT = pl.pallas_call(
        _doc_kernel,
        grid=(B, NSUB),
        in_specs=[
            pl.BlockSpec((1, NE, M), lambda d, j: (d, 0, 0)),
            pl.BlockSpec((1, NR, 2), lambda d, j: (d, 0, 0)),
            pl.BlockSpec((1, L, Hd), lambda d, j: (d, 0, 0)),
            pl.BlockSpec((1, NHS, L, L), lambda d, j: (d, j, 0, 0)),
            pl.BlockSpec((EMB, 1), lambda d, j: (0, 0)),
            pl.BlockSpec((NC, 1), lambda d, j: (0, 0)),
            pl.BlockSpec(memory_space=hbm),
            pl.BlockSpec(memory_space=hbm),
        ],
        out_specs=pl.BlockSpec((B, NC, NR), lambda d, j: (0, 0, 0)),
        out_shape=jax.ShapeDtypeStruct((B, NC, NR), jnp.float32),
        scratch_shapes=[
            pltpu.VMEM((EMB, B * NRP), jnp.bfloat16),
            pltpu.VMEM((EMB, B * NRP), jnp.bfloat16),
            pltpu.VMEM((NE, NE, L), jnp.float32),
            pltpu.VMEM((EMB, 2 * Hd), jnp.float32),
            pltpu.VMEM((NBUF, NC, KW), jnp.float32),
            pltpu.SemaphoreType.DMA((NBUF,)),
            pltpu.SemaphoreType.DMA,
        ],
    )(pos3, hts3, seq_embs, attentions, bh, bb, W_head, W_bil)

    return jnp.transpose(outT, (0, 2, 1)).reshape(TR, NC)


# speedup vs baseline: 1.4470x; 1.0971x over previous
import jax
import jax.numpy as jnp
from jax.experimental import pallas as pl
from jax.experimental.pallas import tpu as pltpu

EMB = 768
BLK = 64
NC = 97
NKB = EMB // BLK
KW = BLK * BLK
NBUF = 3
NSPL = 2


def _wb_copy(wb_hbm, wb_scr, wb_sem, k):
    return pltpu.make_async_copy(
        wb_hbm.at[:, k * KW:(k + 1) * KW], wb_scr.at[k % NBUF],
        wb_sem.at[k % NBUF])


def _doc_kernel(pos_ref, ht_ref, x_ref, a_ref, bh_ref, bb_ref, wh_hbm,
                wb_hbm, out_ref, hs_scr, ts_scr, wh_scr, wb_scr,
                wb_sem, wh_sem):
    f32 = jnp.float32
    bf16 = jnp.bfloat16
    d = pl.program_id(0)
    B = pl.num_programs(0)
    pos = pos_ref[0]
    ht = ht_ref[0]
    X = x_ref[0]
    NE, M = pos.shape
    NR = ht.shape[0]
    L = X.shape[0]
    NH = a_ref.shape[1]
    NRP = hs_scr.shape[1] // B

    @pl.when(d == 0)
    def _prologue():
        hs_scr[...] = jnp.zeros(hs_scr.shape, bf16)
        ts_scr[...] = jnp.zeros(ts_scr.shape, bf16)
        pltpu.make_async_copy(wh_hbm, wh_scr, wh_sem).start()

    @pl.when(d == B - 2)
    def _wb_prefetch():
        for k in range(NBUF):
            _wb_copy(wb_hbm, wb_scr, wb_sem, k).start()

    li = jax.lax.broadcasted_iota(jnp.int32, (NE, M, L), 2)
    S = (pos[:, :, None] == li).astype(bf16).sum(axis=1)

    EX = jnp.exp(X).astype(bf16)
    entT = jnp.log(jax.lax.dot_general(
        EX, S, (((0,), (1,)), ((), ())),
        preferred_element_type=f32))

    Q = jnp.zeros((NE, NE, L), f32)
    for h in range(NH):
        EAh = jax.lax.dot_general(S, a_ref[0, h].astype(bf16),
                                  (((1,), (0,)), ((), ())),
                                  preferred_element_type=f32)
        EAhb = EAh.astype(bf16)
        Q = Q + EAhb[:, None, :] * EAhb[None, :, :]
    Qr = Q.reshape(NE * NE, L).astype(bf16)

    ci = ht[:, 0] * NE + ht[:, 1]
    qi = jax.lax.broadcasted_iota(jnp.int32, (NR, NE * NE), 1)
    OC = (ci[:, None] == qi).astype(bf16)
    rsT = jax.lax.dot_general(Qr, OC, (((0,), (1,)), ((), ())),
                              preferred_element_type=f32)
    rsT = rsT / jnp.sum(rsT, axis=0, keepdims=True)

    rdocT = jax.lax.dot_general(X.astype(bf16), rsT.astype(bf16),
                                (((0,), (0,)), ((), ())),
                                preferred_element_type=f32)

    ei = jax.lax.broadcasted_iota(jnp.int32, (NR, NE), 1)
    OH = (ht[:, 0][:, None] == ei).astype(bf16)
    OT = (ht[:, 1][:, None] == ei).astype(bf16)
    entTb = entT.astype(bf16)
    hsT_e = jax.lax.dot_general(entTb, OH, (((1,), (1,)), ((), ())),
                                preferred_element_type=f32)
    tsT_e = jax.lax.dot_general(entTb, OT, (((1,), (1,)), ((), ())),
                                preferred_element_type=f32)

    @pl.when(d == 0)
    def _wh_wait():
        pltpu.make_async_copy(wh_hbm, wh_scr, wh_sem).wait()
    W1 = wh_scr[:, :EMB].astype(bf16)
    W2 = wh_scr[:, EMB:].astype(bf16)
    b = bh_ref[...]
    ctx = jax.lax.dot_general(W2, rdocT.astype(bf16), (((1,), (0,)), ((), ())),
                              preferred_element_type=f32) + b
    hsT = jnp.tanh(
        jax.lax.dot_general(W1, hsT_e.astype(bf16), (((1,), (0,)), ((), ())),
                            preferred_element_type=f32)
        + ctx).astype(bf16)
    tsT = jnp.tanh(
        jax.lax.dot_general(W1, tsT_e.astype(bf16), (((1,), (0,)), ((), ())),
                            preferred_element_type=f32)
        + ctx).astype(bf16)

    for dd in range(B):
        @pl.when(d == dd)
        def _store(dd=dd):
            hs_scr[:, dd * NRP:dd * NRP + NR] = hsT
            ts_scr[:, dd * NRP:dd * NRP + NR] = tsT

    @pl.when(d == B - 1)
    def _bilinear():
        NT = hs_scr.shape[1]
        NSP = NT // NSPL
        acc = jnp.zeros((NC, NT), f32)
        for k in range(NKB):
            _wb_copy(wb_hbm, wb_scr, wb_sem, k).wait()
            wbk = wb_scr[k % NBUF].astype(bf16)
            parts = []
            for n in range(NSPL):
                hk = hs_scr[k * BLK:(k + 1) * BLK, n * NSP:(n + 1) * NSP]
                tk = ts_scr[k * BLK:(k + 1) * BLK, n * NSP:(n + 1) * NSP]
                b3 = hk[:, None, :] * tk[None, :, :]
                b2 = b3.reshape(KW, NSP)
                parts.append(jax.lax.dot_general(
                    wbk, b2, (((1,), (0,)), ((), ())),
                    preferred_element_type=f32))
            acc = acc + jnp.concatenate(parts, axis=1)
            if k + NBUF < NKB:
                _wb_copy(wb_hbm, wb_scr, wb_sem, k + NBUF).start()
        acc = acc + bb_ref[...]
        for dd in range(B):
            out_ref[dd] = acc[:, dd * NRP:dd * NRP + NR]


def kernel(seq_embs, attentions, entity_pos, hts, n_entities, n_rels,
           W_head, b_head, W_bil, b_bil):
    B, L, Hd = seq_embs.shape
    NH = attentions.shape[1]
    TE = entity_pos.shape[0]
    TR = hts.shape[0]
    NE = TE // B
    M = entity_pos.shape[1]
    NR = TR // B
    NRP = ((NR + 127) // 128) * 128

    pos3 = entity_pos.reshape(B, NE, M)
    hts3 = hts.reshape(B, NR, 2)
    bh = b_head.reshape(EMB, 1)
    bb = b_bil.reshape(NC, 1)

    hbm = pltpu.MemorySpace.HBM
    outT = pl.pallas_call(
        _doc_kernel,
        grid=(B,),
        in_specs=[
            pl.BlockSpec((1, NE, M), lambda d: (d, 0, 0)),
            pl.BlockSpec((1, NR, 2), lambda d: (d, 0, 0)),
            pl.BlockSpec((1, L, Hd), lambda d: (d, 0, 0)),
            pl.BlockSpec((1, NH, L, L), lambda d: (d, 0, 0, 0)),
            pl.BlockSpec((EMB, 1), lambda d: (0, 0)),
            pl.BlockSpec((NC, 1), lambda d: (0, 0)),
            pl.BlockSpec(memory_space=hbm),
            pl.BlockSpec(memory_space=hbm),
        ],
        out_specs=pl.BlockSpec((B, NC, NR), lambda d: (0, 0, 0)),
        out_shape=jax.ShapeDtypeStruct((B, NC, NR), jnp.float32),
        scratch_shapes=[
            pltpu.VMEM((EMB, B * NRP), jnp.bfloat16),
            pltpu.VMEM((EMB, B * NRP), jnp.bfloat16),
            pltpu.VMEM((EMB, 2 * Hd), jnp.float32),
            pltpu.VMEM((NBUF, NC, KW), jnp.float32),
            pltpu.SemaphoreType.DMA((NBUF,)),
            pltpu.SemaphoreType.DMA,
        ],
    )(pos3, hts3, seq_embs, attentions, bh, bb, W_head, W_bil)

    return jnp.transpose(outT, (0, 2, 1)).reshape(TR, NC)
